# baseline (speedup 1.0000x reference)
import functools

import jax
import jax.numpy as jnp
from jax import lax
from jax.experimental import pallas as pl
from jax.experimental.pallas import tpu as pltpu
from jax.experimental.pallas import tpu_sc as plsc

N = 10000
E = 320000
D = 128

NC = 2
NS = 16
NW = NC * NS
EPW = E // NW
C = 104
CHUNKS = EPW // C
CTAIL = EPW - CHUNKS * C
ROWS_PT = 640
RC = 80

_sc_mesh = plsc.VectorSubcoreMesh(
    core_axis_name="c", subcore_axis_name="s", num_cores=NC, num_subcores=NS)


@functools.partial(
    pl.kernel,
    out_type=jax.ShapeDtypeStruct((NC, N, D), jnp.float32),
    mesh=_sc_mesh,
    scratch_types=[
        pltpu.VMEM_SHARED((N, D), jnp.float32),
        pltpu.VMEM((EPW,), jnp.int32),
        pltpu.VMEM((CHUNKS, C), jnp.int32),
        pltpu.VMEM((CTAIL,), jnp.int32),
        pltpu.VMEM((C, D), jnp.float32),
        pltpu.VMEM((C, D), jnp.float32),
        pltpu.SemaphoreType.DMA,
        pltpu.SemaphoreType.DMA,
        pltpu.SemaphoreType.DMA,
        pltpu.SemaphoreType.DMA,
    ],
)
def _sc_segment_sum(hw, srcix, dstix, dsttail, out, acc, src_v, dst_v, dst_t,
                    rows_a, rows_b, gsem_a, gsem_b, ssem_a, ssem_b):
    c = lax.axis_index("c")
    s = lax.axis_index("s")
    wid = c * NS + s
    ebase = pl.multiple_of(wid * EPW, 8)

    pltpu.sync_copy(srcix.at[pl.ds(ebase, EPW)], src_v)
    pltpu.sync_copy(dstix.at[wid], dst_v)

    z = jnp.zeros((16,), jnp.float32)

    def zrow(r, carry):
        for j in range(D // 16):
            rows_a[r, pl.ds(j * 16, 16)] = z
        return carry

    lax.fori_loop(0, RC, zrow, 0)

    rbase = s * ROWS_PT
    nrows = jnp.minimum(ROWS_PT, N - rbase)
    ncopy = nrows // RC

    def zcopy(i, carry):
        off = pl.multiple_of(rbase + i * RC, 8)
        pltpu.sync_copy(rows_a.at[pl.ds(0, RC)], acc.at[pl.ds(off, RC)])
        return carry

    lax.fori_loop(0, ncopy, zcopy, 0)
    plsc.subcore_barrier()

    def prefetch(j, rows, gsem):
        joff = pl.multiple_of(j * C, 8)
        pltpu.async_copy(hw.at[src_v.at[pl.ds(joff, C)]], rows, gsem)

    def consume(j, rows, gsem, ssem):
        pltpu.make_async_copy(hw.at[src_v.at[pl.ds(0, C)]], rows, gsem).wait()
        pltpu.async_copy(rows, acc.at[dst_v.at[j]], ssem, add=True)

    def drain_scatter(rows, ssem):
        pltpu.make_async_copy(hw.at[src_v.at[pl.ds(0, C)]], rows, ssem).wait()

    prefetch(0, rows_a, gsem_a)
    prefetch(1, rows_b, gsem_b)

    def pair(p, carry):
        j = p * 2
        consume(j, rows_a, gsem_a, ssem_a)

        @pl.when(j + 2 < CHUNKS)
        def _():
            drain_scatter(rows_a, ssem_a)
            prefetch(j + 2, rows_a, gsem_a)

        consume(j + 1, rows_b, gsem_b, ssem_b)

        @pl.when(j + 3 < CHUNKS)
        def _():
            drain_scatter(rows_b, ssem_b)
            prefetch(j + 3, rows_b, gsem_b)

        return carry

    lax.fori_loop(0, CHUNKS // 2, pair, 0)
    drain_scatter(rows_a, ssem_a)
    drain_scatter(rows_b, ssem_b)

    toff = pl.multiple_of(ebase + CHUNKS * C, 8)
    pltpu.sync_copy(dsttail.at[pl.ds(toff, CTAIL)], dst_t)
    tj = pl.multiple_of(CHUNKS * C, 8)
    pltpu.sync_copy(hw.at[src_v.at[pl.ds(tj, CTAIL)]],
                    rows_a.at[pl.ds(0, CTAIL)])
    pltpu.sync_copy(rows_a.at[pl.ds(0, CTAIL)], acc.at[dst_t], add=True)
    plsc.subcore_barrier()

    def ocopy(i, carry):
        off = pl.multiple_of(rbase + i * RC, 8)
        pltpu.sync_copy(acc.at[pl.ds(off, RC)], out.at[c, pl.ds(off, RC)])
        return carry

    lax.fori_loop(0, ncopy, ocopy, 0)


BLK = 1000


def _mm_body(h_ref, W_ref, hw_ref):
    hw_ref[...] = jnp.dot(h_ref[...], W_ref[...],
                          preferred_element_type=jnp.float32)


def _res_body(h_ref, Wr_ref, br_ref, res_ref):
    r = jnp.dot(h_ref[...], Wr_ref[...], preferred_element_type=jnp.float32)
    res_ref[...] = jnp.maximum(r + br_ref[...], 0.0)


def _comb_body(agg_ref, b_ref, res_ref, h_ref):
    aggsum = agg_ref[0] + agg_ref[1]
    h_ref[...] = jnp.maximum(aggsum + b_ref[...], 0.0) + res_ref[...]


def _combmm_body(agg_ref, b_ref, res_ref, W_ref, h_ref, hw_ref):
    aggsum = agg_ref[0] + agg_ref[1]
    h = jnp.maximum(aggsum + b_ref[...], 0.0) + res_ref[...]
    h_ref[...] = h
    hw_ref[...] = jnp.dot(h, W_ref[...], preferred_element_type=jnp.float32)


def _mat_spec():
    return pl.BlockSpec((D, D), lambda i: (0, 0))


def _vec_spec():
    return pl.BlockSpec((1, D), lambda i: (0, 0))


def _row_spec():
    return pl.BlockSpec((BLK, D), lambda i: (i, 0))


def _agg_spec():
    return pl.BlockSpec((NC, BLK, D), lambda i: (0, i, 0))


def _f32(n=1):
    s = jax.ShapeDtypeStruct((N, D), jnp.float32)
    return [s] * n if n > 1 else s


def _mm(h, W):
    return pl.pallas_call(
        _mm_body,
        grid=(N // BLK,),
        in_specs=[_row_spec(), _mat_spec()],
        out_specs=_row_spec(),
        out_shape=_f32(),
    )(h, W)


def _res(h, Wr, br):
    return pl.pallas_call(
        _res_body,
        grid=(N // BLK,),
        in_specs=[_row_spec(), _mat_spec(), _vec_spec()],
        out_specs=_row_spec(),
        out_shape=_f32(),
    )(h, Wr, br)


def _combmm(agg, b, res, W):
    return pl.pallas_call(
        _combmm_body,
        grid=(N // BLK,),
        in_specs=[_agg_spec(), _vec_spec(), _row_spec(), _mat_spec()],
        out_specs=[_row_spec(), _row_spec()],
        out_shape=_f32(2),
    )(agg, b, res, W)


def _comb(agg, b, res):
    return pl.pallas_call(
        _comb_body,
        grid=(N // BLK,),
        in_specs=[_agg_spec(), _vec_spec(), _row_spec()],
        out_specs=_row_spec(),
        out_shape=_f32(),
    )(agg, b, res)


@jax.jit
def kernel(feats, edge_index, W1, b1, Wr1, br1, W2, b2, Wr2, br2):
    src = edge_index[0]
    dst = edge_index[1]
    dst3 = dst.reshape(NW, EPW)[:, :CHUNKS * C].reshape(NW, CHUNKS, C)
    b1r = b1.reshape(1, D)
    br1r = br1.reshape(1, D)
    b2r = b2.reshape(1, D)
    br2r = br2.reshape(1, D)

    hw1 = _mm(feats, W1)
    agg1 = _sc_segment_sum(hw1, src, dst3, dst)
    res1 = _res(feats, Wr1, br1r)
    h1, hw2 = _combmm(agg1, b1r, res1, W2)
    agg2 = _sc_segment_sum(hw2, src, dst3, dst)
    res2 = _res(h1, Wr2, br2r)
    return _comb(agg2, b2r, res2)


# speedup vs baseline: 1.0471x; 1.0471x over previous
import functools

import jax
import jax.numpy as jnp
from jax import lax
from jax.experimental import pallas as pl
from jax.experimental.pallas import tpu as pltpu
from jax.experimental.pallas import tpu_sc as plsc

N = 10000
E = 320000
D = 128

NC = 2
NS = 16
NW = NC * NS
EPW = E // NW
C = 128
CHUNKS = EPW // C
CTAIL = EPW - CHUNKS * C
ROWS_PT = 640
RC = 80

_sc_mesh = plsc.VectorSubcoreMesh(
    core_axis_name="c", subcore_axis_name="s", num_cores=NC, num_subcores=NS)


@functools.partial(
    pl.kernel,
    out_type=jax.ShapeDtypeStruct((NC, N, D), jnp.float32),
    mesh=_sc_mesh,
    scratch_types=[
        pltpu.VMEM_SHARED((N, D), jnp.float32),
        pltpu.VMEM((EPW,), jnp.int32),
        pltpu.VMEM((C,), jnp.int32),
        pltpu.VMEM((C,), jnp.int32),
        pltpu.VMEM((CTAIL,), jnp.int32),
        pltpu.VMEM((C, D), jnp.float32),
        pltpu.VMEM((C, D), jnp.float32),
        pltpu.SemaphoreType.DMA,
        pltpu.SemaphoreType.DMA,
        pltpu.SemaphoreType.DMA,
        pltpu.SemaphoreType.DMA,
        pltpu.SemaphoreType.DMA,
        pltpu.SemaphoreType.DMA,
    ],
)
def _sc_segment_sum(hw, srcix, dstix, out, acc, src_v, dst_a, dst_b, dst_t,
                    rows_a, rows_b, gsem_a, gsem_b, isem_a, isem_b, ssem_a,
                    ssem_b):
    c = lax.axis_index("c")
    s = lax.axis_index("s")
    wid = c * NS + s
    ebase = pl.multiple_of(wid * EPW, 8)

    pltpu.sync_copy(srcix.at[pl.ds(ebase, EPW)], src_v)

    z = jnp.zeros((16,), jnp.float32)

    def zrow(r, carry):
        for j in range(D // 16):
            rows_a[r, pl.ds(j * 16, 16)] = z
        return carry

    lax.fori_loop(0, RC, zrow, 0)

    rbase = s * ROWS_PT
    nrows = jnp.minimum(ROWS_PT, N - rbase)
    ncopy = nrows // RC

    def zcopy(i, carry):
        off = pl.multiple_of(rbase + i * RC, 8)
        pltpu.sync_copy(rows_a.at[pl.ds(0, RC)], acc.at[pl.ds(off, RC)])
        return carry

    lax.fori_loop(0, ncopy, zcopy, 0)
    plsc.subcore_barrier()

    def prefetch(j, dstb, rows, gsem, isem):
        off = pl.multiple_of(ebase + j * C, 8)
        pltpu.async_copy(dstix.at[pl.ds(off, C)], dstb, isem)
        joff = pl.multiple_of(j * C, 8)
        pltpu.async_copy(hw.at[src_v.at[pl.ds(joff, C)]], rows, gsem)

    def consume(dstb, rows, gsem, isem, ssem):
        pltpu.make_async_copy(dstix.at[pl.ds(0, C)], dstb, isem).wait()
        pltpu.make_async_copy(hw.at[src_v.at[pl.ds(0, C)]], rows, gsem).wait()
        pltpu.async_copy(rows, acc.at[dstb], ssem, add=True)

    def drain_scatter(rows, ssem):
        pltpu.make_async_copy(hw.at[src_v.at[pl.ds(0, C)]], rows, ssem).wait()

    prefetch(0, dst_a, rows_a, gsem_a, isem_a)
    prefetch(1, dst_b, rows_b, gsem_b, isem_b)

    def pair(p, carry):
        j = p * 2
        consume(dst_a, rows_a, gsem_a, isem_a, ssem_a)

        @pl.when(j + 2 < CHUNKS)
        def _():
            drain_scatter(rows_a, ssem_a)
            prefetch(j + 2, dst_a, rows_a, gsem_a, isem_a)

        consume(dst_b, rows_b, gsem_b, isem_b, ssem_b)

        @pl.when(j + 3 < CHUNKS)
        def _():
            drain_scatter(rows_b, ssem_b)
            prefetch(j + 3, dst_b, rows_b, gsem_b, isem_b)

        return carry

    lax.fori_loop(0, CHUNKS // 2, pair, 0)
    drain_scatter(rows_a, ssem_a)
    drain_scatter(rows_b, ssem_b)

    toff = pl.multiple_of(ebase + CHUNKS * C, 8)
    pltpu.sync_copy(dstix.at[pl.ds(toff, CTAIL)], dst_t)
    tj = pl.multiple_of(CHUNKS * C, 8)
    pltpu.sync_copy(hw.at[src_v.at[pl.ds(tj, CTAIL)]],
                    rows_a.at[pl.ds(0, CTAIL)])
    pltpu.sync_copy(rows_a.at[pl.ds(0, CTAIL)], acc.at[dst_t], add=True)
    plsc.subcore_barrier()

    def ocopy(i, carry):
        off = pl.multiple_of(rbase + i * RC, 8)
        pltpu.sync_copy(acc.at[pl.ds(off, RC)], out.at[c, pl.ds(off, RC)])
        return carry

    lax.fori_loop(0, ncopy, ocopy, 0)


BLK = 1000


def _mm_body(h_ref, W_ref, hw_ref):
    hw_ref[...] = jnp.dot(h_ref[...], W_ref[...],
                          preferred_element_type=jnp.float32)


def _res_body(h_ref, Wr_ref, br_ref, res_ref):
    r = jnp.dot(h_ref[...], Wr_ref[...], preferred_element_type=jnp.float32)
    res_ref[...] = jnp.maximum(r + br_ref[...], 0.0)


def _comb_body(agg_ref, b_ref, res_ref, h_ref):
    aggsum = agg_ref[0] + agg_ref[1]
    h_ref[...] = jnp.maximum(aggsum + b_ref[...], 0.0) + res_ref[...]


def _combmm_body(agg_ref, b_ref, res_ref, W_ref, h_ref, hw_ref):
    aggsum = agg_ref[0] + agg_ref[1]
    h = jnp.maximum(aggsum + b_ref[...], 0.0) + res_ref[...]
    h_ref[...] = h
    hw_ref[...] = jnp.dot(h, W_ref[...], preferred_element_type=jnp.float32)


def _mat_spec():
    return pl.BlockSpec((D, D), lambda i: (0, 0))


def _vec_spec():
    return pl.BlockSpec((1, D), lambda i: (0, 0))


def _row_spec():
    return pl.BlockSpec((BLK, D), lambda i: (i, 0))


def _agg_spec():
    return pl.BlockSpec((NC, BLK, D), lambda i: (0, i, 0))


def _f32(n=1):
    s = jax.ShapeDtypeStruct((N, D), jnp.float32)
    return [s] * n if n > 1 else s


def _mm(h, W):
    return pl.pallas_call(
        _mm_body,
        grid=(N // BLK,),
        in_specs=[_row_spec(), _mat_spec()],
        out_specs=_row_spec(),
        out_shape=_f32(),
    )(h, W)


def _res(h, Wr, br):
    return pl.pallas_call(
        _res_body,
        grid=(N // BLK,),
        in_specs=[_row_spec(), _mat_spec(), _vec_spec()],
        out_specs=_row_spec(),
        out_shape=_f32(),
    )(h, Wr, br)


def _combmm(agg, b, res, W):
    return pl.pallas_call(
        _combmm_body,
        grid=(N // BLK,),
        in_specs=[_agg_spec(), _vec_spec(), _row_spec(), _mat_spec()],
        out_specs=[_row_spec(), _row_spec()],
        out_shape=_f32(2),
    )(agg, b, res, W)


def _comb(agg, b, res):
    return pl.pallas_call(
        _comb_body,
        grid=(N // BLK,),
        in_specs=[_agg_spec(), _vec_spec(), _row_spec()],
        out_specs=_row_spec(),
        out_shape=_f32(),
    )(agg, b, res)


@jax.jit
def kernel(feats, edge_index, W1, b1, Wr1, br1, W2, b2, Wr2, br2):
    src = edge_index[0]
    dst = edge_index[1]
    b1r = b1.reshape(1, D)
    br1r = br1.reshape(1, D)
    b2r = b2.reshape(1, D)
    br2r = br2.reshape(1, D)

    hw1 = _mm(feats, W1)
    agg1 = _sc_segment_sum(hw1, src, dst)
    res1 = _res(feats, Wr1, br1r)
    h1, hw2 = _combmm(agg1, b1r, res1, W2)
    agg2 = _sc_segment_sum(hw2, src, dst)
    res2 = _res(h1, Wr2, br2r)
    return _comb(agg2, b2r, res2)


# speedup vs baseline: 1.0657x; 1.0178x over previous
import functools

import jax
import jax.numpy as jnp
from jax import lax
from jax.experimental import pallas as pl
from jax.experimental.pallas import tpu as pltpu
from jax.experimental.pallas import tpu_sc as plsc

N = 10000
E = 320000
D = 128

NC = 2
NS = 16
NW = NC * NS
EPW = E // NW
C = 128
CHUNKS = EPW // C
CTAIL = EPW - CHUNKS * C
ROWS_PT = 640
RC = 80
ZR = 40

_sc_mesh = plsc.VectorSubcoreMesh(
    core_axis_name="c", subcore_axis_name="s", num_cores=NC, num_subcores=NS)


@functools.partial(
    pl.kernel,
    out_type=jax.ShapeDtypeStruct((NC, N, D), jnp.float32),
    mesh=_sc_mesh,
    scratch_types=[
        pltpu.VMEM_SHARED((N, D), jnp.float32),
        pltpu.VMEM((EPW,), jnp.int32),
        pltpu.VMEM((C,), jnp.int32),
        pltpu.VMEM((C,), jnp.int32),
        pltpu.VMEM((CTAIL,), jnp.int32),
        pltpu.VMEM((C, D), jnp.float32),
        pltpu.VMEM((C, D), jnp.float32),
        pltpu.VMEM((ZR, D), jnp.float32),
        pltpu.SemaphoreType.DMA,
        pltpu.SemaphoreType.DMA,
        pltpu.SemaphoreType.DMA,
        pltpu.SemaphoreType.DMA,
        pltpu.SemaphoreType.DMA,
        pltpu.SemaphoreType.DMA,
    ],
)
def _sc_segment_sum(hw, srcix, dstix, out, acc, src_v, dst_a, dst_b, dst_t,
                    rows_a, rows_b, zbuf, gsem_a, gsem_b, isem_a, isem_b,
                    ssem_a, ssem_b):
    c = lax.axis_index("c")
    s = lax.axis_index("s")
    wid = c * NS + s
    ebase = pl.multiple_of(wid * EPW, 8)

    pltpu.sync_copy(srcix.at[pl.ds(ebase, EPW)], src_v)

    rbase = s * ROWS_PT
    nrows = jnp.minimum(ROWS_PT, N - rbase)

    def prefetch(j, dstb, rows, gsem, isem):
        off = pl.multiple_of(ebase + j * C, 8)
        pltpu.async_copy(dstix.at[pl.ds(off, C)], dstb, isem)
        joff = pl.multiple_of(j * C, 8)
        pltpu.async_copy(hw.at[src_v.at[pl.ds(joff, C)]], rows, gsem)

    def consume(dstb, rows, gsem, isem, ssem):
        pltpu.make_async_copy(dstix.at[pl.ds(0, C)], dstb, isem).wait()
        pltpu.make_async_copy(hw.at[src_v.at[pl.ds(0, C)]], rows, gsem).wait()
        pltpu.async_copy(rows, acc.at[dstb], ssem, add=True)

    def drain_scatter(rows, ssem):
        pltpu.make_async_copy(hw.at[src_v.at[pl.ds(0, C)]], rows, ssem).wait()

    prefetch(0, dst_a, rows_a, gsem_a, isem_a)
    prefetch(1, dst_b, rows_b, gsem_b, isem_b)

    z = jnp.zeros((16,), jnp.float32)

    def zrow(r, carry):
        for j in range(D // 16):
            zbuf[r, pl.ds(j * 16, 16)] = z
        return carry

    lax.fori_loop(0, ZR, zrow, 0)

    def zcopy(i, carry):
        off = pl.multiple_of(rbase + i * ZR, 8)
        pltpu.sync_copy(zbuf, acc.at[pl.ds(off, ZR)])
        return carry

    lax.fori_loop(0, nrows // ZR, zcopy, 0)
    plsc.subcore_barrier()

    def pair(p, carry):
        j = p * 2
        consume(dst_a, rows_a, gsem_a, isem_a, ssem_a)

        @pl.when(j + 2 < CHUNKS)
        def _():
            drain_scatter(rows_a, ssem_a)
            prefetch(j + 2, dst_a, rows_a, gsem_a, isem_a)

        consume(dst_b, rows_b, gsem_b, isem_b, ssem_b)

        @pl.when(j + 3 < CHUNKS)
        def _():
            drain_scatter(rows_b, ssem_b)
            prefetch(j + 3, dst_b, rows_b, gsem_b, isem_b)

        return carry

    lax.fori_loop(0, CHUNKS // 2, pair, 0)
    drain_scatter(rows_a, ssem_a)
    drain_scatter(rows_b, ssem_b)

    toff = pl.multiple_of(ebase + CHUNKS * C, 8)
    pltpu.sync_copy(dstix.at[pl.ds(toff, CTAIL)], dst_t)
    tj = pl.multiple_of(CHUNKS * C, 8)
    pltpu.sync_copy(hw.at[src_v.at[pl.ds(tj, CTAIL)]],
                    rows_a.at[pl.ds(0, CTAIL)])
    pltpu.sync_copy(rows_a.at[pl.ds(0, CTAIL)], acc.at[dst_t], add=True)
    plsc.subcore_barrier()

    def ocopy(i, carry):
        off = pl.multiple_of(rbase + i * RC, 8)
        pltpu.sync_copy(acc.at[pl.ds(off, RC)], out.at[c, pl.ds(off, RC)])
        return carry

    lax.fori_loop(0, nrows // RC, ocopy, 0)


BLK = 1000


def _mm_body(h_ref, W_ref, hw_ref):
    hw_ref[...] = jnp.dot(h_ref[...], W_ref[...],
                          preferred_element_type=jnp.float32)


def _res_body(h_ref, Wr_ref, br_ref, res_ref):
    r = jnp.dot(h_ref[...], Wr_ref[...], preferred_element_type=jnp.float32)
    res_ref[...] = jnp.maximum(r + br_ref[...], 0.0)


def _comb_body(agg_ref, b_ref, res_ref, h_ref):
    aggsum = agg_ref[0] + agg_ref[1]
    h_ref[...] = jnp.maximum(aggsum + b_ref[...], 0.0) + res_ref[...]


def _combmm_body(agg_ref, b_ref, res_ref, W_ref, h_ref, hw_ref):
    aggsum = agg_ref[0] + agg_ref[1]
    h = jnp.maximum(aggsum + b_ref[...], 0.0) + res_ref[...]
    h_ref[...] = h
    hw_ref[...] = jnp.dot(h, W_ref[...], preferred_element_type=jnp.float32)


def _mat_spec():
    return pl.BlockSpec((D, D), lambda i: (0, 0))


def _vec_spec():
    return pl.BlockSpec((1, D), lambda i: (0, 0))


def _row_spec():
    return pl.BlockSpec((BLK, D), lambda i: (i, 0))


def _agg_spec():
    return pl.BlockSpec((NC, BLK, D), lambda i: (0, i, 0))


def _f32(n=1):
    s = jax.ShapeDtypeStruct((N, D), jnp.float32)
    return [s] * n if n > 1 else s


def _mm(h, W):
    return pl.pallas_call(
        _mm_body,
        grid=(N // BLK,),
        in_specs=[_row_spec(), _mat_spec()],
        out_specs=_row_spec(),
        out_shape=_f32(),
    )(h, W)


def _res(h, Wr, br):
    return pl.pallas_call(
        _res_body,
        grid=(N // BLK,),
        in_specs=[_row_spec(), _mat_spec(), _vec_spec()],
        out_specs=_row_spec(),
        out_shape=_f32(),
    )(h, Wr, br)


def _combmm(agg, b, res, W):
    return pl.pallas_call(
        _combmm_body,
        grid=(N // BLK,),
        in_specs=[_agg_spec(), _vec_spec(), _row_spec(), _mat_spec()],
        out_specs=[_row_spec(), _row_spec()],
        out_shape=_f32(2),
    )(agg, b, res, W)


def _comb(agg, b, res):
    return pl.pallas_call(
        _comb_body,
        grid=(N // BLK,),
        in_specs=[_agg_spec(), _vec_spec(), _row_spec()],
        out_specs=_row_spec(),
        out_shape=_f32(),
    )(agg, b, res)


@jax.jit
def kernel(feats, edge_index, W1, b1, Wr1, br1, W2, b2, Wr2, br2):
    src = edge_index[0]
    dst = edge_index[1]
    b1r = b1.reshape(1, D)
    br1r = br1.reshape(1, D)
    b2r = b2.reshape(1, D)
    br2r = br2.reshape(1, D)

    hw1 = _mm(feats, W1)
    agg1 = _sc_segment_sum(hw1, src, dst)
    res1 = _res(feats, Wr1, br1r)
    h1, hw2 = _combmm(agg1, b1r, res1, W2)
    agg2 = _sc_segment_sum(hw2, src, dst)
    res2 = _res(h1, Wr2, br2r)
    return _comb(agg2, b2r, res2)


# speedup vs baseline: 1.0890x; 1.0218x over previous
import functools

import jax
import jax.numpy as jnp
from jax import lax
from jax.experimental import pallas as pl
from jax.experimental.pallas import tpu as pltpu
from jax.experimental.pallas import tpu_sc as plsc

N = 10000
E = 320000
D = 128

NC = 2
NS = 16
NW = NC * NS
EPW = E // NW
C = 128
CHUNKS = EPW // C
CTAIL = EPW - CHUNKS * C
ROWS_PT = 640
RC = 80
ZR = 40

_sc_mesh = plsc.VectorSubcoreMesh(
    core_axis_name="c", subcore_axis_name="s", num_cores=NC, num_subcores=NS)


@functools.partial(
    pl.kernel,
    out_type=jax.ShapeDtypeStruct((NC, N, D), jnp.float32),
    mesh=_sc_mesh,
    scratch_types=[
        pltpu.VMEM_SHARED((N, D), jnp.float32),
        pltpu.VMEM((EPW,), jnp.int32),
        pltpu.VMEM((C,), jnp.int32),
        pltpu.VMEM((C,), jnp.int32),
        pltpu.VMEM((CTAIL,), jnp.int32),
        pltpu.VMEM((C, D), jnp.float32),
        pltpu.VMEM((C, D), jnp.float32),
        pltpu.VMEM((ZR, D), jnp.float32),
        pltpu.SemaphoreType.DMA,
        pltpu.SemaphoreType.DMA,
        pltpu.SemaphoreType.DMA,
        pltpu.SemaphoreType.DMA,
        pltpu.SemaphoreType.DMA,
        pltpu.SemaphoreType.DMA,
    ],
)
def _sc_segment_sum(hw, srcix, dstix, out, acc, src_v, dst_a, dst_b, dst_t,
                    rows_a, rows_b, zbuf, gsem_a, gsem_b, isem_a, isem_b,
                    ssem_a, ssem_b):
    c = lax.axis_index("c")
    s = lax.axis_index("s")
    wid = c * NS + s
    ebase = pl.multiple_of(wid * EPW, 8)

    pltpu.sync_copy(srcix.at[pl.ds(ebase, EPW)], src_v)

    rbase = s * ROWS_PT
    nrows = jnp.minimum(ROWS_PT, N - rbase)

    def prefetch(j, dstb, rows, gsem, isem):
        off = pl.multiple_of(ebase + j * C, 8)
        pltpu.async_copy(dstix.at[pl.ds(off, C)], dstb, isem)
        joff = pl.multiple_of(j * C, 8)
        pltpu.async_copy(hw.at[src_v.at[pl.ds(joff, C)]], rows, gsem)

    def consume(dstb, rows, gsem, isem, ssem):
        pltpu.make_async_copy(dstix.at[pl.ds(0, C)], dstb, isem).wait()
        pltpu.make_async_copy(hw.at[src_v.at[pl.ds(0, C)]], rows, gsem).wait()
        pltpu.async_copy(rows, acc.at[dstb], ssem, add=True)

    def drain_scatter(rows, ssem):
        pltpu.make_async_copy(hw.at[src_v.at[pl.ds(0, C)]], rows, ssem).wait()

    prefetch(0, dst_a, rows_a, gsem_a, isem_a)
    prefetch(1, dst_b, rows_b, gsem_b, isem_b)

    z = jnp.zeros((16,), jnp.float32)

    def zrow(r, carry):
        for j in range(D // 16):
            zbuf[r, pl.ds(j * 16, 16)] = z
        return carry

    lax.fori_loop(0, ZR, zrow, 0)

    def zcopy(i, carry):
        off = pl.multiple_of(rbase + i * ZR, 8)
        pltpu.sync_copy(zbuf, acc.at[pl.ds(off, ZR)])
        return carry

    lax.fori_loop(0, nrows // ZR, zcopy, 0)
    plsc.subcore_barrier()

    def pair(p, carry):
        j = p * 2
        consume(dst_a, rows_a, gsem_a, isem_a, ssem_a)

        @pl.when(j + 2 < CHUNKS)
        def _():
            drain_scatter(rows_a, ssem_a)
            prefetch(j + 2, dst_a, rows_a, gsem_a, isem_a)

        consume(dst_b, rows_b, gsem_b, isem_b, ssem_b)

        @pl.when(j + 3 < CHUNKS)
        def _():
            drain_scatter(rows_b, ssem_b)
            prefetch(j + 3, dst_b, rows_b, gsem_b, isem_b)

        return carry

    lax.fori_loop(0, CHUNKS // 2, pair, 0)
    drain_scatter(rows_a, ssem_a)
    drain_scatter(rows_b, ssem_b)

    toff = pl.multiple_of(ebase + CHUNKS * C, 8)
    pltpu.sync_copy(dstix.at[pl.ds(toff, CTAIL)], dst_t)
    tj = pl.multiple_of(CHUNKS * C, 8)
    pltpu.sync_copy(hw.at[src_v.at[pl.ds(tj, CTAIL)]],
                    rows_a.at[pl.ds(0, CTAIL)])
    pltpu.sync_copy(rows_a.at[pl.ds(0, CTAIL)], acc.at[dst_t], add=True)
    plsc.subcore_barrier()

    def ocopy(i, carry):
        off = pl.multiple_of(rbase + i * RC, 8)
        pltpu.sync_copy(acc.at[pl.ds(off, RC)], out.at[c, pl.ds(off, RC)])
        return carry

    lax.fori_loop(0, nrows // RC, ocopy, 0)


BLK = 2000


def _mm_body(h_ref, W_ref, hw_ref):
    hw_ref[...] = jnp.dot(h_ref[...], W_ref[...],
                          preferred_element_type=jnp.float32)


def _res_body(h_ref, Wr_ref, br_ref, res_ref):
    r = jnp.dot(h_ref[...], Wr_ref[...], preferred_element_type=jnp.float32)
    res_ref[...] = jnp.maximum(r + br_ref[...], 0.0)


def _comb_body(agg_ref, b_ref, res_ref, h_ref):
    aggsum = agg_ref[0] + agg_ref[1]
    h_ref[...] = jnp.maximum(aggsum + b_ref[...], 0.0) + res_ref[...]


def _combmm_body(agg_ref, b_ref, res_ref, W_ref, h_ref, hw_ref):
    aggsum = agg_ref[0] + agg_ref[1]
    h = jnp.maximum(aggsum + b_ref[...], 0.0) + res_ref[...]
    h_ref[...] = h
    hw_ref[...] = jnp.dot(h, W_ref[...], preferred_element_type=jnp.float32)


def _mat_spec():
    return pl.BlockSpec((D, D), lambda i: (0, 0))


def _vec_spec():
    return pl.BlockSpec((1, D), lambda i: (0, 0))


def _row_spec():
    return pl.BlockSpec((BLK, D), lambda i: (i, 0))


def _agg_spec():
    return pl.BlockSpec((NC, BLK, D), lambda i: (0, i, 0))


def _f32(n=1):
    s = jax.ShapeDtypeStruct((N, D), jnp.float32)
    return [s] * n if n > 1 else s


def _mm(h, W):
    return pl.pallas_call(
        _mm_body,
        grid=(N // BLK,),
        in_specs=[_row_spec(), _mat_spec()],
        out_specs=_row_spec(),
        out_shape=_f32(),
    )(h, W)


def _res(h, Wr, br):
    return pl.pallas_call(
        _res_body,
        grid=(N // BLK,),
        in_specs=[_row_spec(), _mat_spec(), _vec_spec()],
        out_specs=_row_spec(),
        out_shape=_f32(),
    )(h, Wr, br)


def _combmm(agg, b, res, W):
    return pl.pallas_call(
        _combmm_body,
        grid=(N // BLK,),
        in_specs=[_agg_spec(), _vec_spec(), _row_spec(), _mat_spec()],
        out_specs=[_row_spec(), _row_spec()],
        out_shape=_f32(2),
    )(agg, b, res, W)


def _comb(agg, b, res):
    return pl.pallas_call(
        _comb_body,
        grid=(N // BLK,),
        in_specs=[_agg_spec(), _vec_spec(), _row_spec()],
        out_specs=_row_spec(),
        out_shape=_f32(),
    )(agg, b, res)


@jax.jit
def kernel(feats, edge_index, W1, b1, Wr1, br1, W2, b2, Wr2, br2):
    src = edge_index[0]
    dst = edge_index[1]
    b1r = b1.reshape(1, D)
    br1r = br1.reshape(1, D)
    b2r = b2.reshape(1, D)
    br2r = br2.reshape(1, D)

    hw1 = _mm(feats, W1)
    agg1 = _sc_segment_sum(hw1, src, dst)
    res1 = _res(feats, Wr1, br1r)
    h1, hw2 = _combmm(agg1, b1r, res1, W2)
    agg2 = _sc_segment_sum(hw2, src, dst)
    res2 = _res(h1, Wr2, br2r)
    return _comb(agg2, b2r, res2)


# speedup vs baseline: 1.1103x; 1.0195x over previous
import functools

import jax
import jax.numpy as jnp
from jax import lax
from jax.experimental import pallas as pl
from jax.experimental.pallas import tpu as pltpu
from jax.experimental.pallas import tpu_sc as plsc

N = 10000
E = 320000
D = 128

NC = 2
NS = 16
NW = NC * NS
EPW = E // NW
C = 128
CHUNKS = EPW // C
CTAIL = EPW - CHUNKS * C
ROWS_PT = 640
RC = 80
ZR = 40

_sc_mesh = plsc.VectorSubcoreMesh(
    core_axis_name="c", subcore_axis_name="s", num_cores=NC, num_subcores=NS)


@functools.partial(
    pl.kernel,
    out_type=jax.ShapeDtypeStruct((NC, N, D), jnp.float32),
    mesh=_sc_mesh,
    scratch_types=[
        pltpu.VMEM_SHARED((N, D), jnp.float32),
        pltpu.VMEM((EPW,), jnp.int32),
        pltpu.VMEM((C,), jnp.int32),
        pltpu.VMEM((C,), jnp.int32),
        pltpu.VMEM((CTAIL,), jnp.int32),
        pltpu.VMEM((C, D), jnp.float32),
        pltpu.VMEM((C, D), jnp.float32),
        pltpu.VMEM((ZR, D), jnp.float32),
        pltpu.SemaphoreType.DMA,
        pltpu.SemaphoreType.DMA,
        pltpu.SemaphoreType.DMA,
        pltpu.SemaphoreType.DMA,
        pltpu.SemaphoreType.DMA,
        pltpu.SemaphoreType.DMA,
    ],
)
def _sc_segment_sum(hw, srcix, dstix, out, acc, src_v, dst_a, dst_b, dst_t,
                    rows_a, rows_b, zbuf, gsem_a, gsem_b, isem_a, isem_b,
                    ssem_a, ssem_b):
    c = lax.axis_index("c")
    s = lax.axis_index("s")
    wid = c * NS + s
    ebase = pl.multiple_of(wid * EPW, 8)

    pltpu.sync_copy(srcix.at[pl.ds(ebase, EPW)], src_v)

    rbase = s * ROWS_PT
    nrows = jnp.minimum(ROWS_PT, N - rbase)

    def prefetch(j, dstb, rows, gsem, isem):
        off = pl.multiple_of(ebase + j * C, 8)
        pltpu.async_copy(dstix.at[pl.ds(off, C)], dstb, isem)
        joff = pl.multiple_of(j * C, 8)
        pltpu.async_copy(hw.at[src_v.at[pl.ds(joff, C)]], rows, gsem)

    def consume(dstb, rows, gsem, isem, ssem):
        pltpu.make_async_copy(dstix.at[pl.ds(0, C)], dstb, isem).wait()
        pltpu.make_async_copy(hw.at[src_v.at[pl.ds(0, C)]], rows, gsem).wait()
        pltpu.async_copy(rows, acc.at[dstb], ssem, add=True)

    def drain_scatter(rows, ssem):
        pltpu.make_async_copy(hw.at[src_v.at[pl.ds(0, C)]], rows, ssem).wait()

    prefetch(0, dst_a, rows_a, gsem_a, isem_a)
    prefetch(1, dst_b, rows_b, gsem_b, isem_b)

    z = jnp.zeros((16,), jnp.float32)

    def zrow(r, carry):
        for j in range(D // 16):
            zbuf[r, pl.ds(j * 16, 16)] = z
        return carry

    lax.fori_loop(0, ZR, zrow, 0)

    def zcopy(i, carry):
        off = pl.multiple_of(rbase + i * ZR, 8)
        pltpu.sync_copy(zbuf, acc.at[pl.ds(off, ZR)])
        return carry

    lax.fori_loop(0, nrows // ZR, zcopy, 0)
    plsc.subcore_barrier()

    def pair(p, carry):
        j = p * 2
        consume(dst_a, rows_a, gsem_a, isem_a, ssem_a)

        @pl.when(j + 2 < CHUNKS)
        def _():
            drain_scatter(rows_a, ssem_a)
            prefetch(j + 2, dst_a, rows_a, gsem_a, isem_a)

        consume(dst_b, rows_b, gsem_b, isem_b, ssem_b)

        @pl.when(j + 3 < CHUNKS)
        def _():
            drain_scatter(rows_b, ssem_b)
            prefetch(j + 3, dst_b, rows_b, gsem_b, isem_b)

        return carry

    lax.fori_loop(0, CHUNKS // 2, pair, 0)
    drain_scatter(rows_a, ssem_a)
    drain_scatter(rows_b, ssem_b)

    toff = pl.multiple_of(ebase + CHUNKS * C, 8)
    pltpu.sync_copy(dstix.at[pl.ds(toff, CTAIL)], dst_t)
    tj = pl.multiple_of(CHUNKS * C, 8)
    pltpu.sync_copy(hw.at[src_v.at[pl.ds(tj, CTAIL)]],
                    rows_a.at[pl.ds(0, CTAIL)])
    pltpu.sync_copy(rows_a.at[pl.ds(0, CTAIL)], acc.at[dst_t], add=True)
    plsc.subcore_barrier()

    def ocopy(i, carry):
        off = pl.multiple_of(rbase + i * RC, 8)
        pltpu.sync_copy(acc.at[pl.ds(off, RC)], out.at[c, pl.ds(off, RC)])
        return carry

    lax.fori_loop(0, nrows // RC, ocopy, 0)


BLK = 5000


def _mm_body(h_ref, W_ref, hw_ref):
    hw_ref[...] = jnp.dot(h_ref[...], W_ref[...],
                          preferred_element_type=jnp.float32)


def _res_body(h_ref, Wr_ref, br_ref, res_ref):
    r = jnp.dot(h_ref[...], Wr_ref[...], preferred_element_type=jnp.float32)
    res_ref[...] = jnp.maximum(r + br_ref[...], 0.0)


def _comb_body(agg_ref, b_ref, res_ref, h_ref):
    aggsum = agg_ref[0] + agg_ref[1]
    h_ref[...] = jnp.maximum(aggsum + b_ref[...], 0.0) + res_ref[...]


def _combmm_body(agg_ref, b_ref, res_ref, W_ref, h_ref, hw_ref):
    aggsum = agg_ref[0] + agg_ref[1]
    h = jnp.maximum(aggsum + b_ref[...], 0.0) + res_ref[...]
    h_ref[...] = h
    hw_ref[...] = jnp.dot(h, W_ref[...], preferred_element_type=jnp.float32)


def _mat_spec():
    return pl.BlockSpec((D, D), lambda i: (0, 0))


def _vec_spec():
    return pl.BlockSpec((1, D), lambda i: (0, 0))


def _row_spec():
    return pl.BlockSpec((BLK, D), lambda i: (i, 0))


def _agg_spec():
    return pl.BlockSpec((NC, BLK, D), lambda i: (0, i, 0))


def _f32(n=1):
    s = jax.ShapeDtypeStruct((N, D), jnp.float32)
    return [s] * n if n > 1 else s


def _mm(h, W):
    return pl.pallas_call(
        _mm_body,
        grid=(N // BLK,),
        in_specs=[_row_spec(), _mat_spec()],
        out_specs=_row_spec(),
        out_shape=_f32(),
    )(h, W)


def _res(h, Wr, br):
    return pl.pallas_call(
        _res_body,
        grid=(N // BLK,),
        in_specs=[_row_spec(), _mat_spec(), _vec_spec()],
        out_specs=_row_spec(),
        out_shape=_f32(),
    )(h, Wr, br)


def _combmm(agg, b, res, W):
    return pl.pallas_call(
        _combmm_body,
        grid=(N // BLK,),
        in_specs=[_agg_spec(), _vec_spec(), _row_spec(), _mat_spec()],
        out_specs=[_row_spec(), _row_spec()],
        out_shape=_f32(2),
    )(agg, b, res, W)


def _comb(agg, b, res):
    return pl.pallas_call(
        _comb_body,
        grid=(N // BLK,),
        in_specs=[_agg_spec(), _vec_spec(), _row_spec()],
        out_specs=_row_spec(),
        out_shape=_f32(),
    )(agg, b, res)


@jax.jit
def kernel(feats, edge_index, W1, b1, Wr1, br1, W2, b2, Wr2, br2):
    src = edge_index[0]
    dst = edge_index[1]
    b1r = b1.reshape(1, D)
    br1r = br1.reshape(1, D)
    b2r = b2.reshape(1, D)
    br2r = br2.reshape(1, D)

    hw1 = _mm(feats, W1)
    agg1 = _sc_segment_sum(hw1, src, dst)
    res1 = _res(feats, Wr1, br1r)
    h1, hw2 = _combmm(agg1, b1r, res1, W2)
    agg2 = _sc_segment_sum(hw2, src, dst)
    res2 = _res(h1, Wr2, br2r)
    return _comb(agg2, b2r, res2)
